# baseline (speedup 1.0000x reference)
import functools

import jax
import jax.numpy as jnp
from jax import lax
from jax.experimental import pallas as pl
from jax.experimental.pallas import tpu as pltpu
from jax.experimental.pallas import tpu_sc as plsc

B, L, H = 4, 2048, 1024
N = B * L
NC, NS = 2, 16
NW = NC * NS
P = L // NW
PC = 8
NCH = P // PC
CR = B * PC
LANES = 16


def kernel(X, W_word, W_pos):
    idx = (X.reshape(B, NW, NCH, PC)
            .transpose(1, 2, 0, 3)
            .reshape(N)
            .astype(jnp.int32))
    mesh = plsc.VectorSubcoreMesh(core_axis_name="c", subcore_axis_name="s")

    @functools.partial(
        pl.kernel,
        out_type=jax.ShapeDtypeStruct((N, H), jnp.float32),
        mesh=mesh,
        scratch_types=[
            pltpu.VMEM((B * P,), jnp.int32),
            pltpu.VMEM((CR, H), jnp.float32),
            pltpu.VMEM((CR, H), jnp.float32),
            pltpu.VMEM((CR, H), jnp.float32),
            pltpu.VMEM((PC, H), jnp.float32),
            pltpu.VMEM((PC, H), jnp.float32),
            pltpu.SemaphoreType.DMA,
            pltpu.SemaphoreType.DMA,
            pltpu.SemaphoreType.DMA,
            pltpu.SemaphoreType.DMA,
            pltpu.SemaphoreType.DMA,
            pltpu.SemaphoreType.DMA,
            pltpu.SemaphoreType.DMA,
            pltpu.SemaphoreType.DMA,
        ],
    )
    def embed(w_hbm, p_hbm, i_hbm, o_hbm,
              idx_v, ring0, ring1, ring2, pos0, pos1,
              sg0, sg1, sg2, so0, so1, so2, sp0, sp1):
        ring = [ring0, ring1, ring2]
        pos = [pos0, pos1]
        sg = [sg0, sg1, sg2]
        so = [so0, so1, so2]
        sp = [sp0, sp1]

        wid = lax.axis_index("s") * NC + lax.axis_index("c")
        l0 = wid * P

        pltpu.sync_copy(i_hbm.at[pl.ds(wid * (B * P), B * P)], idx_v)

        def start(k):
            p = k % 3
            g = pltpu.async_copy(
                w_hbm.at[idx_v.at[pl.ds(k * CR, CR)]], ring[p], sg[p])
            q = pltpu.async_copy(
                p_hbm.at[pl.ds(l0 + k * PC, PC)], pos[k % 2], sp[k % 2])
            return g, q

        inflight = {0: start(0), 1: start(1)}
        out_cp = {}

        for k in range(NCH):
            p = k % 3
            g, q = inflight.pop(k)
            g.wait()
            q.wait()

            @plsc.parallel_loop(0, H, step=LANES, unroll=2)
            def _(c):
                s = pl.ds(c, LANES)
                for r in range(PC):
                    pv = pos[k % 2].at[r, s][...]
                    for b in range(B):
                        ring[p].at[b * PC + r, s][...] = jnp.maximum(
                            ring[p].at[b * PC + r, s][...] + pv, 0.0)

            for b in range(B):
                out_cp[(k, b)] = pltpu.async_copy(
                    ring[p].at[pl.ds(b * PC, PC)],
                    o_hbm.at[pl.ds(b * L + l0 + k * PC, PC)],
                    so[p])

            if k + 2 < NCH:
                if k - 1 >= 0:
                    for b in range(B):
                        out_cp.pop((k - 1, b)).wait()
                inflight[k + 2] = start(k + 2)

        for kk in (NCH - 2, NCH - 1):
            for b in range(B):
                out_cp.pop((kk, b)).wait()

    out = embed(W_word, W_pos, idx)
    return out.reshape(B, L, H)


# speedup vs baseline: 1.1882x; 1.1882x over previous
import functools

import jax
import jax.numpy as jnp
from jax import lax
from jax.experimental import pallas as pl
from jax.experimental.pallas import tpu as pltpu
from jax.experimental.pallas import tpu_sc as plsc

B, L, H = 4, 2048, 1024
N = B * L
NC, NS = 2, 16
NW = NC * NS
P = L // NW
PC = 8
NCH = P // PC
CR = B * PC
LANES = 16


def kernel(X, W_word, W_pos):
    idx = (X.reshape(B, NW, NCH, PC)
            .transpose(1, 2, 0, 3)
            .reshape(N)
            .astype(jnp.int32))
    mesh = plsc.VectorSubcoreMesh(core_axis_name="c", subcore_axis_name="s")

    @functools.partial(
        pl.kernel,
        out_type=jax.ShapeDtypeStruct((N, H), jnp.float32),
        mesh=mesh,
        scratch_types=[
            pltpu.VMEM((B * P,), jnp.int32),
            pltpu.VMEM((CR, H), jnp.float32),
            pltpu.VMEM((CR, H), jnp.float32),
            pltpu.VMEM((CR, H), jnp.float32),
            pltpu.VMEM((PC, H), jnp.float32),
            pltpu.VMEM((PC, H), jnp.float32),
            pltpu.SemaphoreType.DMA,
            pltpu.SemaphoreType.DMA,
            pltpu.SemaphoreType.DMA,
            pltpu.SemaphoreType.DMA,
            pltpu.SemaphoreType.DMA,
            pltpu.SemaphoreType.DMA,
            pltpu.SemaphoreType.DMA,
            pltpu.SemaphoreType.DMA,
        ],
    )
    def embed(w_hbm, p_hbm, i_hbm, o_hbm,
              idx_v, ring0, ring1, ring2, pos0, pos1,
              sg0, sg1, sg2, so0, so1, so2, sp0, sp1):
        ring = [ring0, ring1, ring2]
        pos = [pos0, pos1]
        sg = [sg0, sg1, sg2]
        so = [so0, so1, so2]
        sp = [sp0, sp1]

        wid = lax.axis_index("s") * NC + lax.axis_index("c")
        l0 = wid * P

        pltpu.sync_copy(i_hbm.at[pl.ds(wid * (B * P), B * P)], idx_v)

        def start(k):
            p = k % 3
            g = pltpu.async_copy(
                w_hbm.at[idx_v.at[pl.ds(k * CR, CR)]], ring[p], sg[p])
            q = pltpu.async_copy(
                p_hbm.at[pl.ds(l0 + k * PC, PC)], pos[k % 2], sp[k % 2])
            return g, q

        inflight = {0: start(0), 1: start(1)}
        out_cp = {}

        for k in range(NCH):
            p = k % 3
            g, q = inflight.pop(k)
            g.wait()
            q.wait()

            @pl.loop(0, PC)
            def _(r):
                @plsc.parallel_loop(0, H, step=LANES, unroll=8)
                def _(c):
                    s = pl.ds(c, LANES)
                    pv = pos[k % 2].at[r, s][...]
                    for b in range(B):
                        ring[p].at[b * PC + r, s][...] = jnp.maximum(
                            ring[p].at[b * PC + r, s][...] + pv, 0.0)

            for b in range(B):
                out_cp[(k, b)] = pltpu.async_copy(
                    ring[p].at[pl.ds(b * PC, PC)],
                    o_hbm.at[pl.ds(b * L + l0 + k * PC, PC)],
                    so[p])

            if k + 2 < NCH:
                if k - 1 >= 0:
                    for b in range(B):
                        out_cp.pop((k - 1, b)).wait()
                inflight[k + 2] = start(k + 2)

        for kk in (NCH - 2, NCH - 1):
            for b in range(B):
                out_cp.pop((kk, b)).wait()

    out = embed(W_word, W_pos, idx)
    return out.reshape(B, L, H)
